# Initial kernel scaffold; baseline (speedup 1.0000x reference)
import functools

import jax
import jax.numpy as jnp
from jax import lax
from jax.experimental import pallas as pl
from jax.experimental.pallas import tpu as pltpu
from jax.experimental.pallas import tpu_sc as plsc

B = 16384
F = 26
D = 16
NC, NS, L = 2, 16, 16
NW = NC * NS
RPW = B // NW
IPW = RPW * F
XW = 128
XROWS_W = IPW // XW
C = 64
NCHUNK = RPW // C
GJ = C * F // XW
GROUPS = C // L


def _sc_body(x_hbm, emb_hbm, un_hbm, z_hbm,
             idx_v, rows_v0, rows_v1, u_v0, u_v1, usum_v, out_v,
             sem0, sem1):
    wid = lax.axis_index("s") * NC + lax.axis_index("c")
    rows_bufs = (rows_v0, rows_v1)
    u_bufs = (u_v0, u_v1)
    sems = (sem0, sem1)

    pltpu.sync_copy(x_hbm.at[pl.ds(wid * XROWS_W, XROWS_W)], idx_v)

    def issue(c):
        slot = c % 2
        descs = []
        for j in range(GJ):
            row = c * GJ + j
            descs.append(pltpu.async_copy(
                emb_hbm.at[idx_v.at[row]],
                rows_bufs[slot].at[pl.ds(j * XW, XW)], sems[slot]))
            descs.append(pltpu.async_copy(
                un_hbm.at[idx_v.at[row]],
                u_bufs[slot].at[pl.ds(j * XW, XW)], sems[slot]))
        return descs

    lane = lax.iota(jnp.int32, L)
    zeros16 = lane * 0

    descs = issue(0)
    for c in range(NCHUNK):
        nxt = issue(c + 1) if c + 1 < NCHUNK else []
        for dsc in descs:
            dsc.wait()
        descs = nxt
        slot = c % 2
        rows_b = rows_bufs[slot]
        u_b = u_bufs[slot]

        def ubody(g, carry):
            base = g * (L * F)
            acc = jnp.zeros((L,), jnp.float32)
            for f in range(F):
                vals = plsc.load_gather(u_b, [base + lane * F + f, zeros16])
                acc = acc + vals
            usum_v[pl.ds(g * L, L)] = acc
            return carry
        lax.fori_loop(0, GROUPS, ubody, 0)

        def rbody(r, carry):
            acc = jnp.zeros((D,), jnp.float32)
            acc2 = jnp.zeros((D,), jnp.float32)
            for f in range(F):
                v = rows_b[r * F + f, :]
                acc = acc + v
                acc2 = acc2 + v * v
            out_v[r, :] = acc * acc - acc2 + usum_v[r]
            return carry
        lax.fori_loop(0, C, rbody, 0)

        pltpu.sync_copy(out_v, z_hbm.at[pl.ds(wid * RPW + c * C, C)])


_sc_ffm = functools.partial(
    pl.kernel,
    out_type=jax.ShapeDtypeStruct((B, D), jnp.float32),
    mesh=plsc.VectorSubcoreMesh(core_axis_name="c", subcore_axis_name="s"),
    scratch_types=[
        pltpu.VMEM((XROWS_W, XW), jnp.int32),
        pltpu.VMEM((C * F, D), jnp.float32),
        pltpu.VMEM((C * F, D), jnp.float32),
        pltpu.VMEM((C * F, 1), jnp.float32),
        pltpu.VMEM((C * F, 1), jnp.float32),
        pltpu.VMEM((C,), jnp.float32),
        pltpu.VMEM((C, D), jnp.float32),
        pltpu.SemaphoreType.DMA,
        pltpu.SemaphoreType.DMA,
    ],
)(_sc_body)


def _logsig_body(z_ref, o_ref):
    z = z_ref[...]
    o_ref[...] = jnp.where(z >= 0.0,
                           -jnp.log1p(jnp.exp(-z)),
                           z - jnp.log1p(jnp.exp(z)))


def _logsig(z):
    z2 = z.reshape(B * D // 128, 128)
    out = pl.pallas_call(
        _logsig_body,
        out_shape=jax.ShapeDtypeStruct(z2.shape, jnp.float32),
    )(z2)
    return out.reshape(B, D)


def kernel(X, emb_table, unary_table):
    x2d = X.reshape(B * F // XW, XW)
    z = _sc_ffm(x2d, emb_table, unary_table)
    return _logsig(z)


# speedup vs baseline: 1.3254x; 1.3254x over previous
import functools

import jax
import jax.numpy as jnp
from jax import lax
from jax.experimental import pallas as pl
from jax.experimental.pallas import tpu as pltpu
from jax.experimental.pallas import tpu_sc as plsc

B = 16384
F = 26
D = 16
NC, NS, L = 2, 16, 16
NW = NC * NS
RPW = B // NW
IPW = RPW * F
XW = 128
XROWS_W = IPW // XW
C = 64
NCHUNK = RPW // C
GJ = C * F // XW
GROUPS = C // L


def _sc_body(x_hbm, emb_hbm, un_hbm, z_hbm,
             idx_v, rows_v0, rows_v1, u_v0, u_v1, usum_v, out_v,
             sem0, sem1):
    wid = lax.axis_index("s") * NC + lax.axis_index("c")
    rows_bufs = (rows_v0, rows_v1)
    u_bufs = (u_v0, u_v1)
    sems = (sem0, sem1)

    pltpu.sync_copy(x_hbm.at[pl.ds(wid * XROWS_W, XROWS_W)], idx_v)

    def issue(c):
        slot = c % 2
        descs = []
        for j in range(GJ):
            row = c * GJ + j
            descs.append(pltpu.async_copy(
                emb_hbm.at[idx_v.at[row]],
                rows_bufs[slot].at[pl.ds(j * XW, XW)], sems[slot]))
            descs.append(pltpu.async_copy(
                un_hbm.at[idx_v.at[row]],
                u_bufs[slot].at[pl.ds(j * XW, XW)], sems[slot]))
        return descs

    lane = lax.iota(jnp.int32, L)
    zeros16 = lane * 0

    descs = issue(0)
    for c in range(NCHUNK):
        nxt = issue(c + 1) if c + 1 < NCHUNK else []
        for dsc in descs:
            dsc.wait()
        descs = nxt
        slot = c % 2
        rows_b = rows_bufs[slot]
        u_b = u_bufs[slot]

        def ubody(g, carry):
            base = g * (L * F)
            acc = jnp.zeros((L,), jnp.float32)
            for f in range(F):
                vals = plsc.load_gather(u_b, [base + lane * F + f])
                acc = acc + vals
            for i in range(L):
                usum_v[g * L + i, :] = jnp.broadcast_to(acc[i], (D,))
            return carry
        lax.fori_loop(0, GROUPS, ubody, 0)

        def rbody(r, carry):
            acc = jnp.zeros((D,), jnp.float32)
            acc2 = jnp.zeros((D,), jnp.float32)
            for f in range(F):
                v = rows_b[r * F + f, :]
                acc = acc + v
                acc2 = acc2 + v * v
            out_v[r, :] = acc * acc - acc2 + usum_v[r, :]
            return carry
        lax.fori_loop(0, C, rbody, 0)

        pltpu.sync_copy(out_v, z_hbm.at[pl.ds(wid * RPW + c * C, C)])


_sc_ffm = functools.partial(
    pl.kernel,
    out_type=jax.ShapeDtypeStruct((B, D), jnp.float32),
    mesh=plsc.VectorSubcoreMesh(core_axis_name="c", subcore_axis_name="s"),
    scratch_types=[
        pltpu.VMEM((XROWS_W, XW), jnp.int32),
        pltpu.VMEM((C * F, D), jnp.float32),
        pltpu.VMEM((C * F, D), jnp.float32),
        pltpu.VMEM((C * F,), jnp.float32),
        pltpu.VMEM((C * F,), jnp.float32),
        pltpu.VMEM((C, D), jnp.float32),
        pltpu.VMEM((C, D), jnp.float32),
        pltpu.SemaphoreType.DMA,
        pltpu.SemaphoreType.DMA,
    ],
    compiler_params=pltpu.CompilerParams(needs_layout_passes=False,
                                         use_tc_tiling_on_sc=False),
)(_sc_body)


def _logsig_body(z_ref, o_ref):
    z = z_ref[...]
    o_ref[...] = jnp.where(z >= 0.0,
                           -jnp.log1p(jnp.exp(-z)),
                           z - jnp.log1p(jnp.exp(z)))


def _logsig(z):
    z2 = z.reshape(B * D // 128, 128)
    out = pl.pallas_call(
        _logsig_body,
        out_shape=jax.ShapeDtypeStruct(z2.shape, jnp.float32),
    )(z2)
    return out.reshape(B, D)


def kernel(X, emb_table, unary_table):
    x2d = X.reshape(B * F // XW, XW)
    z = _sc_ffm(x2d, emb_table, unary_table.reshape(-1))
    return _logsig(z)
